# Initial kernel scaffold; baseline (speedup 1.0000x reference)
import jax
import jax.numpy as jnp
from jax.experimental import pallas as pl


def kernel(sent, mask, word_embed, mask_embed):
    raise NotImplementedError("write your pallas kernel here")


# speedup vs baseline: 2.8008x; 2.8008x over previous
import jax
import jax.numpy as jnp
from jax import lax
from jax.experimental import pallas as pl
from jax.experimental.pallas import tpu as pltpu
from jax.experimental.pallas import tpu_sc as plsc

EMBED_DIM = 128
BATCH = 4096
MAX_LEN = 50

NC = 2
NS = 16
NW = NC * NS
ROWS_PER_W = BATCH // NW
G = 8
IDX_PER_G = G * MAX_LEN
CH = 80
NCH = IDX_PER_G // CH
NGROUP = ROWS_PER_W // G
ACC_ROWS = 144
GARBAGE = 132


def _sc_kernel(sent_hbm, mask_hbm, table_hbm, out_vec_hbm, out_avg_hbm,
               idx_v, mask_v, dst_v, rows_v, zero_v, acc_v, avg_v, cnts_v,
               accum_sh, sem):
    c = lax.axis_index("c")
    s = lax.axis_index("s")
    wid = c * NS + s
    sbase = s * ACC_ROWS

    zvec = jnp.zeros((16,), jnp.float32)
    for r in range(16):
        for k in range(EMBED_DIM // 16):
            zero_v[r, pl.ds(16 * k, 16)] = zvec
    for t in range(ROWS_PER_W // 16):
        pltpu.sync_copy(zero_v, accum_sh.at[pl.ds(sbase + 16 * t, 16)])

    def group_body(j, carry):
        grp = wid * NGROUP + j
        base = grp * IDX_PER_G
        brow = grp * G

        pltpu.sync_copy(sent_hbm.at[grp], idx_v)
        pltpu.sync_copy(mask_hbm.at[pl.ds(base, IDX_PER_G)],
                        mask_v.at[pl.ds(0, IDX_PER_G)])

        gathers = []
        for r in range(NCH):
            gathers.append(pltpu.async_copy(
                table_hbm.at[idx_v.at[r]],
                rows_v.at[pl.ds(CH * r, CH)], sem))

        live0 = sbase + j * G
        for r in range(NCH):
            for cc in range(CH // 16):
                off = CH * r + 16 * cc
                i_vec = lax.iota(jnp.int32, 16) + off
                g_vec = i_vec // MAX_LEN
                m = mask_v[pl.ds(off, 16)]
                garbage = sbase + GARBAGE + (i_vec & 3)
                dst_v[r, pl.ds(16 * cc, 16)] = jnp.where(
                    m > 0, live0 + g_vec, garbage)
        for g in range(G):
            cnt_vec = jnp.zeros((16,), jnp.int32)
            for t in range(4):
                off = g * MAX_LEN + 16 * t
                pos = lax.iota(jnp.int32, 16) + off
                m = mask_v[pl.ds(off, 16)]
                cnt_vec = cnt_vec + jnp.where(pos < (g + 1) * MAX_LEN, m, 0)
            cntf = jnp.sum(cnt_vec, axis=0).astype(jnp.float32)
            cnts_v[j * G + g, pl.ds(0, 16)] = zvec + cntf

        for cp in gathers:
            cp.wait()

        pltpu.sync_copy(rows_v, out_vec_hbm.at[pl.ds(brow * MAX_LEN,
                                                     IDX_PER_G)])

        for r in range(NCH):
            pltpu.sync_copy(rows_v.at[pl.ds(CH * r, CH)],
                            accum_sh.at[dst_v.at[r]], add=True)
        return carry

    lax.fori_loop(0, NGROUP, group_body, 0)

    def avg_body(jj, carry):
        pltpu.sync_copy(accum_sh.at[pl.ds(sbase + jj * G, G)], acc_v)
        for g in range(G):
            cvec = cnts_v[jj * G + g, pl.ds(0, 16)]
            for k in range(EMBED_DIM // 16):
                avg_v[g, pl.ds(16 * k, 16)] = (
                    acc_v[g, pl.ds(16 * k, 16)] / cvec)
        pltpu.sync_copy(avg_v,
                        out_avg_hbm.at[pl.ds(wid * ROWS_PER_W + jj * G, G)])
        return carry

    lax.fori_loop(0, NGROUP, avg_body, 0)


@jax.jit
def _run(sent3d, mask_flat, word_embed):
    mesh = plsc.VectorSubcoreMesh(core_axis_name="c", subcore_axis_name="s")
    fn = pl.kernel(
        _sc_kernel,
        out_type=(
            jax.ShapeDtypeStruct((BATCH * MAX_LEN, EMBED_DIM), jnp.float32),
            jax.ShapeDtypeStruct((BATCH, EMBED_DIM), jnp.float32),
        ),
        mesh=mesh,
        compiler_params=pltpu.CompilerParams(needs_layout_passes=False),
        scratch_types=[
            pltpu.VMEM((NCH, CH), jnp.int32),
            pltpu.VMEM((IDX_PER_G + 16,), jnp.int32),
            pltpu.VMEM((NCH, CH), jnp.int32),
            pltpu.VMEM((IDX_PER_G, EMBED_DIM), jnp.float32),
            pltpu.VMEM((16, EMBED_DIM), jnp.float32),
            pltpu.VMEM((G, EMBED_DIM), jnp.float32),
            pltpu.VMEM((G, EMBED_DIM), jnp.float32),
            pltpu.VMEM((ROWS_PER_W, 16), jnp.float32),
            pltpu.VMEM_SHARED((NS * ACC_ROWS, EMBED_DIM), jnp.float32),
            pltpu.SemaphoreType.DMA,
        ],
        name="glove_mask_avg",
    )
    return fn(sent3d, mask_flat, word_embed)


def kernel(sent, mask, word_embed, mask_embed):
    sent3d = sent.reshape(BATCH * MAX_LEN // IDX_PER_G, NCH, CH).astype(
        jnp.int32)
    mask_flat = mask.reshape(BATCH * MAX_LEN).astype(jnp.int32)
    out_vec, out_avg = _run(sent3d, mask_flat, word_embed)
    return out_vec.reshape(BATCH, MAX_LEN, EMBED_DIM), out_avg


# speedup vs baseline: 3.3185x; 1.1848x over previous
import jax
import jax.numpy as jnp
from jax import lax
from jax.experimental import pallas as pl
from jax.experimental.pallas import tpu as pltpu
from jax.experimental.pallas import tpu_sc as plsc

EMBED_DIM = 128
BATCH = 4096
MAX_LEN = 50

NC = 2
NS = 16
NW = NC * NS
ROWS_PER_W = BATCH // NW
G = 8
IDX_PER_G = G * MAX_LEN
CH = 80
NCH = IDX_PER_G // CH
NGROUP = ROWS_PER_W // G
NCHUNK = NGROUP * NCH
NBUF = 4
ACC_ROWS = 144
GARBAGE = 132


def _sc_kernel(sent_hbm, mask_hbm, table_hbm, out_vec_hbm, out_avg_hbm,
               idx_v0, idx_v1, mask_v0, mask_v1,
               dst_v0, dst_v1, dst_v2, dst_v3,
               rows_v0, rows_v1, rows_v2, rows_v3,
               zero_v, acc_v, avg_v, cnts_v,
               accum_sh, sem_in, sem_g, sem_out, sem_sc):
    c = lax.axis_index("c")
    s = lax.axis_index("s")
    wid = c * NS + s
    sbase = s * ACC_ROWS
    idx_b = (idx_v0, idx_v1)
    mask_b = (mask_v0, mask_v1)
    dst_b = (dst_v0, dst_v1, dst_v2, dst_v3)
    rows_b = (rows_v0, rows_v1, rows_v2, rows_v3)

    zvec = jnp.zeros((16,), jnp.float32)
    for r in range(16):
        for k in range(EMBED_DIM // 16):
            zero_v[r, pl.ds(16 * k, 16)] = zvec
    for t in range(ROWS_PER_W // 16):
        pltpu.sync_copy(zero_v, accum_sh.at[pl.ds(sbase + 16 * t, 16)])

    def fetch(j):
        grp = wid * NGROUP + j
        jb = j & 1
        pltpu.async_copy(sent_hbm.at[grp], idx_b[jb], sem_in)
        pltpu.async_copy(mask_hbm.at[pl.ds(grp * IDX_PER_G, IDX_PER_G)],
                         mask_b[jb].at[pl.ds(0, IDX_PER_G)], sem_in)

    def wait_fetch(j):
        grp = wid * NGROUP + j
        jb = j & 1
        pltpu.make_async_copy(sent_hbm.at[grp], idx_b[jb], sem_in).wait()
        pltpu.make_async_copy(
            mask_hbm.at[pl.ds(grp * IDX_PER_G, IDX_PER_G)],
            mask_b[jb].at[pl.ds(0, IDX_PER_G)], sem_in).wait()

    def gather_copy(k):
        j, r = k // NCH, k % NCH
        return pltpu.make_async_copy(
            table_hbm.at[idx_b[j & 1].at[r]], rows_b[k % NBUF], sem_g)

    def write_copy(k):
        grp0 = wid * NGROUP * NCH
        return pltpu.make_async_copy(
            rows_b[k % NBUF],
            out_vec_hbm.at[pl.ds((grp0 + k) * CH, CH)], sem_out)

    def scatter_copy(k):
        return pltpu.make_async_copy(
            rows_b[k % NBUF], accum_sh.at[dst_b[k % NBUF].at[0]], sem_sc)

    def fire_scatter(k):
        pltpu.async_copy(rows_b[k % NBUF],
                         accum_sh.at[dst_b[k % NBUF].at[0]], sem_sc,
                         add=True)

    def compute_dst(k):
        j, r = k // NCH, k % NCH
        live0 = sbase + j * G
        for cc in range(CH // 16):
            off = CH * r + 16 * cc
            i_vec = lax.iota(jnp.int32, 16) + off
            g_vec = i_vec // MAX_LEN
            m = mask_b[j & 1][pl.ds(off, 16)]
            garbage = sbase + GARBAGE + (i_vec & 3)
            dst_b[k % NBUF][0, pl.ds(16 * cc, 16)] = jnp.where(
                m > 0, live0 + g_vec, garbage)

    def compute_counts(j):
        for g in range(G):
            cnt_vec = jnp.zeros((16,), jnp.int32)
            for t in range(4):
                off = g * MAX_LEN + 16 * t
                pos = lax.iota(jnp.int32, 16) + off
                m = mask_b[j & 1][pl.ds(off, 16)]
                cnt_vec = cnt_vec + jnp.where(pos < (g + 1) * MAX_LEN, m, 0)
            cntf = jnp.sum(cnt_vec, axis=0).astype(jnp.float32)
            cnts_v[j * G + g, pl.ds(0, 16)] = zvec + cntf

    fetch(0)
    wait_fetch(0)
    fetch(1)
    for k in range(NBUF - 1):
        gather_copy(k).start()
    for k in range(NCHUNK):
        j, r = k // NCH, k % NCH
        gather_copy(k).wait()
        if r == 0:
            compute_counts(j)
        compute_dst(k)
        if k > 0:
            write_copy(k - 1).wait()
            scatter_copy(k - 1).wait()
        write_copy(k).start()
        fire_scatter(k)
        if r == NCH - 1 and j + 2 < NGROUP:
            fetch(j + 2)
        kk = k + NBUF - 1
        if kk < NCHUNK:
            jj = kk // NCH
            if kk % NCH == 0 and jj > 0:
                wait_fetch(jj)
            gather_copy(kk).start()
    write_copy(NCHUNK - 1).wait()
    scatter_copy(NCHUNK - 1).wait()

    def avg_body(jj, carry):
        pltpu.sync_copy(accum_sh.at[pl.ds(sbase + jj * G, G)], acc_v)
        for g in range(G):
            cvec = cnts_v[jj * G + g, pl.ds(0, 16)]
            for k in range(EMBED_DIM // 16):
                avg_v[g, pl.ds(16 * k, 16)] = (
                    acc_v[g, pl.ds(16 * k, 16)] / cvec)
        pltpu.sync_copy(
            avg_v, out_avg_hbm.at[pl.ds(wid * ROWS_PER_W + jj * G, G)])
        return carry

    lax.fori_loop(0, NGROUP, avg_body, 0)


@jax.jit
def _run(sent3d, mask_flat, word_embed):
    mesh = plsc.VectorSubcoreMesh(core_axis_name="c", subcore_axis_name="s")
    fn = pl.kernel(
        _sc_kernel,
        out_type=(
            jax.ShapeDtypeStruct((BATCH * MAX_LEN, EMBED_DIM), jnp.float32),
            jax.ShapeDtypeStruct((BATCH, EMBED_DIM), jnp.float32),
        ),
        mesh=mesh,
        compiler_params=pltpu.CompilerParams(needs_layout_passes=False),
        scratch_types=[
            pltpu.VMEM((NCH, CH), jnp.int32),
            pltpu.VMEM((NCH, CH), jnp.int32),
            pltpu.VMEM((IDX_PER_G + 16,), jnp.int32),
            pltpu.VMEM((IDX_PER_G + 16,), jnp.int32),
            pltpu.VMEM((1, CH), jnp.int32),
            pltpu.VMEM((1, CH), jnp.int32),
            pltpu.VMEM((1, CH), jnp.int32),
            pltpu.VMEM((1, CH), jnp.int32),
            pltpu.VMEM((CH, EMBED_DIM), jnp.float32),
            pltpu.VMEM((CH, EMBED_DIM), jnp.float32),
            pltpu.VMEM((CH, EMBED_DIM), jnp.float32),
            pltpu.VMEM((CH, EMBED_DIM), jnp.float32),
            pltpu.VMEM((16, EMBED_DIM), jnp.float32),
            pltpu.VMEM((G, EMBED_DIM), jnp.float32),
            pltpu.VMEM((G, EMBED_DIM), jnp.float32),
            pltpu.VMEM((ROWS_PER_W, 16), jnp.float32),
            pltpu.VMEM_SHARED((NS * ACC_ROWS, EMBED_DIM), jnp.float32),
            pltpu.SemaphoreType.DMA,
            pltpu.SemaphoreType.DMA,
            pltpu.SemaphoreType.DMA,
            pltpu.SemaphoreType.DMA,
        ],
        name="glove_mask_avg",
    )
    return fn(sent3d, mask_flat, word_embed)


def kernel(sent, mask, word_embed, mask_embed):
    sent3d = sent.reshape(BATCH * MAX_LEN // IDX_PER_G, NCH, CH).astype(
        jnp.int32)
    mask_flat = mask.reshape(BATCH * MAX_LEN).astype(jnp.int32)
    out_vec, out_avg = _run(sent3d, mask_flat, word_embed)
    return out_vec.reshape(BATCH, MAX_LEN, EMBED_DIM), out_avg
